# baseline (speedup 1.0000x reference)
import jax
import jax.numpy as jnp
from jax import lax
from jax.experimental import pallas as pl
from jax.experimental.pallas import tpu as pltpu

_B, _C, _H, _W = 64, 96, 64, 64
_BB = 2


def _body(lab_ref, pk_ref, out_ref):
    step = pl.program_id(0)
    iota = lax.broadcasted_iota(jnp.int32, (_C, _H, _W), 2)
    rev = (_W - 1) - iota

    for i in range(_BB):
        blk = pk_ref[i]
        x = blk[:, :, :_W]
        y = blk[:, :, _W:]
        r = lab_ref[step * _BB + i]

        idx1 = jnp.where(r == 0, iota, rev)
        idxy = jnp.where(r == 2, rev, iota)

        xt = jnp.swapaxes(x, 1, 2)
        a = jnp.where((r == 1) | (r == 2), xt, x)
        bb = jnp.take_along_axis(a, idx1, axis=2)
        c = jnp.swapaxes(bb, 1, 2)
        xr = jnp.where(r <= 1, bb, c)
        yg = jnp.take_along_axis(y, idxy, axis=2)

        diff = xr - yg
        out_ref[0, i, 0] = jnp.sum(diff * diff)
        out_ref[0, i, 1] = jnp.sum(xr * jnp.log(xr / jnp.maximum(yg, 1e-9)))


def kernel(hp, hp_rot, label_rot):
    packed = jnp.concatenate([hp, hp_rot], axis=3)
    grid_spec = pltpu.PrefetchScalarGridSpec(
        num_scalar_prefetch=1,
        grid=(_B // _BB,),
        in_specs=[
            pl.BlockSpec((_BB, _C, _H, 2 * _W), lambda b, lab: (b, 0, 0, 0)),
        ],
        out_specs=[
            pl.BlockSpec(memory_space=pltpu.SMEM, block_shape=(1, _BB, 2),
                         index_map=lambda b, lab: (b, 0, 0)),
        ],
    )
    out = pl.pallas_call(
        _body,
        grid_spec=grid_spec,
        out_shape=[
            jax.ShapeDtypeStruct((_B // _BB, _BB, 2), jnp.float32),
        ],
    )(label_rot.astype(jnp.int32), packed)[0]
    kl_s = out[:, :, 1].sum() / _B
    l2_s = out[:, :, 0].sum() / (_B * _C * _H * _W)
    return kl_s * 0.4 + l2_s * 0.6


# speedup vs baseline: 1.1044x; 1.1044x over previous
import jax
import jax.numpy as jnp
from jax import lax
from jax.experimental import pallas as pl
from jax.experimental.pallas import tpu as pltpu

_B, _C, _H, _W = 64, 96, 64, 64
_BB = 2


def _body(lab_ref, hp_ref, hprot_ref, out_ref):
    step = pl.program_id(0)
    rev = jnp.broadcast_to(
        (_W - 1) - lax.broadcasted_iota(jnp.int32, (_C, _H, _W), 2),
        (_C, _H, _W))

    def _g(v):
        return jnp.take_along_axis(v, rev, axis=2)

    for i in range(_BB):
        x = hp_ref[i]
        y = hprot_ref[i]
        r = lab_ref[step * _BB + i]

        xt = jnp.swapaxes(x, 1, 2)
        a = jnp.where((r == 1) | (r == 2), xt, x)
        g = _g(a)
        b = jnp.where(r == 0, a, g)
        c = jnp.swapaxes(b, 1, 2)
        xr = jnp.where(r <= 1, b, c)
        yg = jnp.where(r == 2, _g(y), y)

        diff = xr - yg
        out_ref[0, i, 0] = jnp.sum(diff * diff)
        out_ref[0, i, 1] = jnp.sum(xr * jnp.log(xr / jnp.maximum(yg, 1e-9)))


def kernel(hp, hp_rot, label_rot):
    grid_spec = pltpu.PrefetchScalarGridSpec(
        num_scalar_prefetch=1,
        grid=(_B // _BB,),
        in_specs=[
            pl.BlockSpec((_BB, _C, _H, _W), lambda b, lab: (b, 0, 0, 0)),
            pl.BlockSpec((_BB, _C, _H, _W), lambda b, lab: (b, 0, 0, 0)),
        ],
        out_specs=[
            pl.BlockSpec(memory_space=pltpu.SMEM, block_shape=(1, _BB, 2),
                         index_map=lambda b, lab: (b, 0, 0)),
        ],
    )
    out = pl.pallas_call(
        _body,
        grid_spec=grid_spec,
        out_shape=[
            jax.ShapeDtypeStruct((_B // _BB, _BB, 2), jnp.float32),
        ],
    )(label_rot.astype(jnp.int32), hp, hp_rot)[0]
    kl_s = out[:, :, 1].sum() / _B
    l2_s = out[:, :, 0].sum() / (_B * _C * _H * _W)
    return kl_s * 0.4 + l2_s * 0.6
